# baseline (speedup 1.0000x reference)
import jax
import jax.numpy as jnp
from jax import lax
from jax.experimental import pallas as pl
from jax.experimental.pallas import tpu as pltpu
from jax.experimental.pallas import tpu_sc as plsc

BS = 4096
NUM_AXIOMS = 200
N = BS * NUM_AXIOMS
EMBED_DIM = 64
EPAD = 128

NC = 2
NS = 16
NW = NC * NS
PW = N // NW
CHUNK = 64
ROWS = 2 * CHUNK
NSTREAM = ROWS // 128
IDXROWS = 2 * PW // 128
NCHUNK = PW // CHUNK
RING = 4


def _body(emb_hbm, xi_hbm, out_hbm,
          idx_v, rows0, rows1, rows2, rows3, sc0, sc1, sc2, sc3,
          isem, gsem0, gsem1, gsem2, gsem3, osem0, osem1, osem2, osem3):
    rows = (rows0, rows1, rows2, rows3)
    sc = (sc0, sc1, sc2, sc3)
    gsem = (gsem0, gsem1, gsem2, gsem3)
    osem = (osem0, osem1, osem2, osem3)

    wid = lax.axis_index("c") * NS + lax.axis_index("s")
    lanes = lax.iota(jnp.int32, 16)
    lane15 = lanes == 15

    pltpu.async_copy(
        xi_hbm.at[pl.ds(wid * IDXROWS, IDXROWS)], idx_v, isem).wait()

    def gather_start(c, b):
        for j in range(NSTREAM):
            pltpu.async_copy(
                emb_hbm.at[idx_v.at[c * NSTREAM + j]],
                rows[b].at[pl.ds(j * 128, 128)], gsem[b])

    def gather_wait(b):
        for j in range(NSTREAM):
            pltpu.make_async_copy(
                emb_hbm.at[idx_v.at[j]],
                rows[b].at[pl.ds(j * 128, 128)], gsem[b]).wait()

    def out_start(c, b):
        pltpu.async_copy(
            sc[b], out_hbm.at[pl.ds(wid * PW + c * CHUNK, CHUNK)], osem[b])

    def out_wait(b):
        pltpu.make_async_copy(
            sc[b], out_hbm.at[pl.ds(0, CHUNK)], osem[b]).wait()

    def compute(b):
        rb, sb = rows[b], sc[b]

        def pair_body(p, _):
            prods = []
            for q in range(4):
                lv = rb[2 * p, pl.ds(q * 16, 16)]
                rv = rb[2 * p + 1, pl.ds(q * 16, 16)]
                prods.append(lv * rv)
            part = (prods[0] + prods[1]) + (prods[2] + prods[3])
            cum = plsc.cumsum(part)
            idx = jnp.full((16,), p, jnp.int32)
            plsc.store_scatter(sb, [idx], cum, mask=lane15)
            return _

        lax.fori_loop(0, CHUNK, pair_body, 0, unroll=4)

    for b in range(RING):
        gather_start(b, b)

    def ring_body(i, _):
        for b in range(RING):
            c = i * RING + b
            gather_wait(b)
            out_wait(b)
            compute(b)
            out_start(c, b)
            gather_start((c + RING) % NCHUNK, b)
        return _

    for b in range(RING):
        gather_wait(b)
        compute(b)
        out_start(b, b)
        gather_start(b + RING, b)

    lax.fori_loop(1, NCHUNK // RING, ring_body, 0, unroll=False)

    for b in range(RING):
        gather_wait(b)
        out_wait(b)


@jax.jit
def _scores(xi2, emb):
    mesh = plsc.VectorSubcoreMesh(
        core_axis_name="c", subcore_axis_name="s",
        num_cores=NC, num_subcores=NS)
    f = pl.kernel(
        _body,
        out_type=jax.ShapeDtypeStruct((N,), jnp.float32),
        mesh=mesh,
        scratch_types=[
            pltpu.VMEM((IDXROWS, 128), jnp.int32),
            pltpu.VMEM((ROWS, EPAD), jnp.float32),
            pltpu.VMEM((ROWS, EPAD), jnp.float32),
            pltpu.VMEM((ROWS, EPAD), jnp.float32),
            pltpu.VMEM((ROWS, EPAD), jnp.float32),
            pltpu.VMEM((CHUNK,), jnp.float32),
            pltpu.VMEM((CHUNK,), jnp.float32),
            pltpu.VMEM((CHUNK,), jnp.float32),
            pltpu.VMEM((CHUNK,), jnp.float32),
            pltpu.SemaphoreType.DMA,
            pltpu.SemaphoreType.DMA,
            pltpu.SemaphoreType.DMA,
            pltpu.SemaphoreType.DMA,
            pltpu.SemaphoreType.DMA,
            pltpu.SemaphoreType.DMA,
            pltpu.SemaphoreType.DMA,
            pltpu.SemaphoreType.DMA,
            pltpu.SemaphoreType.DMA,
        ],
        compiler_params=pltpu.CompilerParams(
            needs_layout_passes=False, use_tc_tiling_on_sc=False),
    )
    return f(emb, xi2)


def kernel(x, emb):
    bs, num_axioms, ents = x.shape
    xi2 = x.reshape(-1).astype(jnp.int32).reshape(2 * N // 128, 128)
    embp = jnp.pad(emb, ((0, 0), (0, EPAD - EMBED_DIM)))
    scores = _scores(xi2, embp)
    return scores.reshape(bs, num_axioms)
